# baseline (speedup 1.0000x reference)
import functools
import math

import jax
import jax.numpy as jnp
from jax import lax
from jax.experimental import pallas as pl
from jax.experimental.pallas import tpu as pltpu
from jax.experimental.pallas import tpu_sc as plsc

D_MODEL = 64
SCALE = math.sqrt(D_MODEL)

_NC = 2
_NS = 16
_NW = _NC * _NS

N_BUF = 4
LEAD = 2
ROW_UNROLL = 4
L = 16


def _make_kernel(R, S):
    assert R % _NW == 0
    r_per_w = R // _NW
    assert r_per_w % N_BUF == 0 and r_per_w > N_BUF
    assert S % ROW_UNROLL == 0 and S % 8 == 0
    i_per_w = r_per_w * S
    n_vec = (S + L - 1) // L
    s_pad = n_vec * L

    mesh = plsc.VectorSubcoreMesh(core_axis_name="c", subcore_axis_name="s")

    @functools.partial(
        pl.kernel,
        mesh=mesh,
        out_type=jax.ShapeDtypeStruct((R, S, 2 * D_MODEL), jnp.float32),
        compiler_params=pltpu.CompilerParams(use_tc_tiling_on_sc=True),
        scratch_types=(
            [pltpu.VMEM((i_per_w + s_pad,), jnp.int32)]
            + [pltpu.VMEM((s_pad,), jnp.int32) for _ in range(N_BUF)]
            + [pltpu.VMEM((S, 2 * D_MODEL), jnp.float32) for _ in range(N_BUF)]
            + [pltpu.SemaphoreType.DMA for _ in range(2 * N_BUF)]
        ),
    )
    def emb(x_hbm, tbl2_hbm, out_hbm, x_all, *rest):
        pidx = rest[:N_BUF]
        gbuf = rest[N_BUF:2 * N_BUF]
        gsem = rest[2 * N_BUF:3 * N_BUF]
        ssem = rest[3 * N_BUF:]

        wid = lax.axis_index("s") * _NC + lax.axis_index("c")
        rbase = wid * r_per_w

        def mk_gather(g, b):
            return pltpu.make_async_copy(
                tbl2_hbm.at[pidx[b].at[pl.ds(0, S)]], gbuf[b], gsem[b])

        def mk_store(g, b):
            return pltpu.make_async_copy(
                gbuf[b], out_hbm.at[rbase + g], ssem[b])

        def stage_pidx(g, b):
            for k in range(n_vec):
                v = x_all[pl.ds(g * S + k * L, L)]
                pidx[b][pl.ds(k * L, L)] = lax.shift_right_logical(v, 1)

        pltpu.sync_copy(x_hbm.at[pl.ds(wid * i_per_w, i_per_w)],
                        x_all.at[pl.ds(0, i_per_w)])
        for b in range(LEAD):
            stage_pidx(b, b)
            mk_gather(b, b).start()

        def outer(i, carry):
            for b in range(N_BUF):
                g = i * N_BUF + b
                bb = (b + LEAD) % N_BUF

                @pl.when(g + LEAD < r_per_w)
                def _issue():
                    @pl.when(g >= LEAD)
                    def _drain():
                        mk_store(g - LEAD, bb).wait()
                    stage_pidx(g + LEAD, bb)
                    mk_gather(g + LEAD, bb).start()

                mk_gather(g, b).wait()

                buf = gbuf[b]

                def row_body(r, c):
                    for u in range(ROW_UNROLL):
                        rr = r * ROW_UNROLL + u
                        par = x_all[pl.ds(g * S + rr, L)][0] & 1
                        off = par * D_MODEL
                        for c4 in range(D_MODEL // L):
                            src = buf[rr, pl.ds(off + c4 * L, L)] * SCALE
                            buf[rr, pl.ds(c4 * L, L)] = src
                    return c

                lax.fori_loop(0, S // ROW_UNROLL, row_body, 0)
                mk_store(g, b).start()
            return carry

        lax.fori_loop(0, r_per_w // N_BUF, outer, 0)
        for g in range(r_per_w - N_BUF, r_per_w):
            mk_store(g, g % N_BUF).wait()

    return emb


def kernel(x, table):
    R, S = x.shape
    flat_x = x.reshape(R * S)
    tbl2 = table.reshape(table.shape[0] // 2, 2 * table.shape[1])
    padded = _make_kernel(R, S)(flat_x, tbl2)
    return padded[:, :, :D_MODEL]


# speedup vs baseline: 1.5960x; 1.5960x over previous
import functools
import math

import jax
import jax.numpy as jnp
from jax import lax
from jax.experimental import pallas as pl
from jax.experimental.pallas import tpu as pltpu
from jax.experimental.pallas import tpu_sc as plsc

D_MODEL = 64
SCALE = math.sqrt(D_MODEL)

_NC = 2
_NS = 16
_NW = _NC * _NS

N_BUF = 4
LEAD = 2
ROW_UNROLL = 4
L = 16


def _make_kernel(R, S):
    assert R % _NW == 0
    r_per_w = R // _NW
    assert r_per_w % N_BUF == 0 and r_per_w > N_BUF
    assert S % ROW_UNROLL == 0 and S % 8 == 0
    i_per_w = r_per_w * S

    mesh = plsc.VectorSubcoreMesh(core_axis_name="c", subcore_axis_name="s")

    @functools.partial(
        pl.kernel,
        mesh=mesh,
        out_type=jax.ShapeDtypeStruct((R, S, 2 * D_MODEL), jnp.float32),
        compiler_params=pltpu.CompilerParams(use_tc_tiling_on_sc=True),
        scratch_types=(
            [pltpu.VMEM((i_per_w,), jnp.int32)]
            + [pltpu.VMEM((S, 2 * D_MODEL), jnp.float32) for _ in range(N_BUF)]
            + [pltpu.SemaphoreType.DMA for _ in range(2 * N_BUF)]
        ),
    )
    def emb(x_hbm, tbl_hbm, out_hbm, x_all, *rest):
        gbuf = rest[:N_BUF]
        gsem = rest[N_BUF:2 * N_BUF]
        ssem = rest[2 * N_BUF:]

        wid = lax.axis_index("s") * _NC + lax.axis_index("c")
        rbase = wid * r_per_w

        def mk_gather(g, b):
            return pltpu.make_async_copy(
                tbl_hbm.at[x_all.at[pl.ds(g * S, S)]], gbuf[b], gsem[b])

        def mk_store(g, b):
            return pltpu.make_async_copy(
                gbuf[b], out_hbm.at[rbase + g], ssem[b])

        pltpu.sync_copy(x_hbm.at[pl.ds(wid * i_per_w, i_per_w)], x_all)
        for b in range(LEAD):
            mk_gather(b, b).start()

        def outer(i, carry):
            for b in range(N_BUF):
                g = i * N_BUF + b
                bb = (b + LEAD) % N_BUF

                @pl.when(g + LEAD < r_per_w)
                def _issue():
                    @pl.when(g >= LEAD)
                    def _drain():
                        mk_store(g - LEAD, bb).wait()
                    mk_gather(g + LEAD, bb).start()

                mk_gather(g, b).wait()

                buf = gbuf[b]

                def row_body(r, c):
                    for u in range(ROW_UNROLL):
                        rr = r * ROW_UNROLL + u
                        for c4 in range(D_MODEL // L):
                            sl = pl.ds(c4 * L, L)
                            buf[rr, sl] = buf[rr, sl] * SCALE
                    return c

                lax.fori_loop(0, S // ROW_UNROLL, row_body, 0)
                mk_store(g, b).start()
            return carry

        lax.fori_loop(0, r_per_w // N_BUF, outer, 0)
        for g in range(r_per_w - N_BUF, r_per_w):
            mk_store(g, g % N_BUF).wait()

    return emb


def kernel(x, table):
    R, S = x.shape
    flat_x = x.reshape(R * S)
    wide_tbl = jnp.pad(table, ((0, 0), (0, D_MODEL)))
    padded = _make_kernel(R, S)(flat_x, wide_tbl)
    return padded[:, :, :D_MODEL]
